# Initial kernel scaffold; baseline (speedup 1.0000x reference)
import jax
import jax.numpy as jnp
from jax.experimental import pallas as pl


def kernel(x, pairlist, f_ij, rcut_ij, W1, b1, Wf1, bf1, Wf2, bf2, Wo1, bo1, Wo2, bo2):
    raise NotImplementedError("write your pallas kernel here")


# speedup vs baseline: 2.2412x; 2.2412x over previous
import functools

import jax
import jax.numpy as jnp
from jax import lax
from jax.experimental import pallas as pl
from jax.experimental.pallas import tpu as pltpu
from jax.experimental.pallas import tpu_sc as plsc

_LOG2 = 0.6931471805599453

_NC = 2
_NS = 16
_NW = _NC * _NS


def _ssp(t):
    return jnp.maximum(t, 0.0) + jnp.log1p(jnp.exp(-jnp.abs(t))) - _LOG2


def _h_body(x_ref, w_ref, b_ref, o_ref):
    o_ref[...] = (
        jnp.dot(x_ref[...], w_ref[...], preferred_element_type=jnp.float32)
        + b_ref[...]
    )


def _wij_body(f_ref, rc_ref, w1_ref, b1_ref, w2_ref, b2_ref, o_ref):
    u = jnp.dot(f_ref[...], w1_ref[...], preferred_element_type=jnp.float32)
    t = _ssp(u + b1_ref[...])
    w = jnp.dot(t, w2_ref[...], preferred_element_type=jnp.float32) + b2_ref[...]
    o_ref[...] = w * rc_ref[...]


def _out_body(a_ref, w1_ref, b1_ref, w2_ref, b2_ref, o_ref):
    a = a_ref[0] + a_ref[1]
    t = _ssp(
        jnp.dot(a, w1_ref[...], preferred_element_type=jnp.float32) + b1_ref[...]
    )
    o_ref[...] = (
        jnp.dot(t, w2_ref[...], preferred_element_type=jnp.float32) + b2_ref[...]
    )


def _make_sc_scatter(N, E, D, C, interpret=False):
    EPW = E // _NW
    CH = EPW // C
    SUP = 25 if CH % 25 == 0 else CH
    RPT = N // _NS
    assert EPW % C == 0 and N % _NS == 0 and D % 16 == 0
    assert RPT % 8 == 0 and RPT % C == 0 and CH % SUP == 0

    mesh = plsc.VectorSubcoreMesh(
        core_axis_name="c", subcore_axis_name="s",
        num_cores=_NC, num_subcores=_NS,
    )

    @functools.partial(
        pl.kernel,
        out_type=jax.ShapeDtypeStruct((_NC, N, D), jnp.float32),
        mesh=mesh,
        interpret=interpret,
        scratch_types=[
            pltpu.VMEM((SUP, C), jnp.int32),
            pltpu.VMEM((SUP, C), jnp.int32),
            pltpu.VMEM((C, D), jnp.float32),
            pltpu.VMEM((C, D), jnp.float32),
            pltpu.VMEM_SHARED((N, D), jnp.float32),
            pltpu.SemaphoreType.DMA,
        ],
    )
    def sc_scatter(h_hbm, wij_hbm, idxj_hbm, idxi_hbm, out_hbm,
                   idxj_v, idxi_v, xj_v, wij_v, acc_sh, sem):
        c = lax.axis_index("c")
        s = lax.axis_index("s")
        w = s * _NC + c

        def _zrow(i, carry):
            for k in range(D // 16):
                wij_v[i, pl.ds(k * 16, 16)] = jnp.zeros((16,), jnp.float32)
            return carry
        lax.fori_loop(0, C, _zrow, 0)
        for z in range(RPT // C):
            pltpu.sync_copy(wij_v, acc_sh.at[pl.ds(s * RPT + z * C, C)])
        plsc.subcore_barrier()

        base_w = w * EPW

        def _super(g, carry):
            pltpu.sync_copy(idxj_hbm.at[w, g], idxj_v)
            pltpu.sync_copy(idxi_hbm.at[w, g], idxi_v)

            def _chunk(j, cc):
                base = base_w + (g * SUP + j) * C
                pltpu.async_copy(h_hbm.at[idxj_v.at[j]], xj_v, sem).wait()
                pltpu.sync_copy(wij_hbm.at[pl.ds(base, C)], wij_v)

                def _mrow(e, ccc):
                    for k in range(D // 16):
                        sl = pl.ds(k * 16, 16)
                        xj_v[e, sl] = xj_v[e, sl] * wij_v[e, sl]
                    return ccc
                lax.fori_loop(0, C, _mrow, 0)

                pltpu.sync_copy(xj_v, acc_sh.at[idxi_v.at[j]], add=True)
                return cc
            lax.fori_loop(0, SUP, _chunk, 0)
            return carry
        lax.fori_loop(0, CH // SUP, _super, 0)

        plsc.subcore_barrier()
        pltpu.sync_copy(acc_sh.at[pl.ds(s * RPT, RPT)],
                        out_hbm.at[c, pl.ds(s * RPT, RPT)])

    return sc_scatter


def kernel(x, pairlist, f_ij, rcut_ij, W1, b1, Wf1, bf1, Wf2, bf2, Wo1, bo1,
           Wo2, bo2):
    N, D = x.shape
    E, R = f_ij.shape
    F = W1.shape[1]
    BN = 2000 if N % 2000 == 0 else N
    BE = 2000 if E % 2000 == 0 else E
    C = 80

    h = pl.pallas_call(
        _h_body,
        grid=(N // BN,),
        in_specs=[
            pl.BlockSpec((BN, D), lambda i: (i, 0)),
            pl.BlockSpec((D, F), lambda i: (0, 0)),
            pl.BlockSpec((1, F), lambda i: (0, 0)),
        ],
        out_specs=pl.BlockSpec((BN, F), lambda i: (i, 0)),
        out_shape=jax.ShapeDtypeStruct((N, F), jnp.float32),
    )(x, W1, b1.reshape(1, F))

    wij = pl.pallas_call(
        _wij_body,
        grid=(E // BE,),
        in_specs=[
            pl.BlockSpec((BE, R), lambda i: (i, 0)),
            pl.BlockSpec((BE, 1), lambda i: (i, 0)),
            pl.BlockSpec((R, F), lambda i: (0, 0)),
            pl.BlockSpec((1, F), lambda i: (0, 0)),
            pl.BlockSpec((F, F), lambda i: (0, 0)),
            pl.BlockSpec((1, F), lambda i: (0, 0)),
        ],
        out_specs=pl.BlockSpec((BE, F), lambda i: (i, 0)),
        out_shape=jax.ShapeDtypeStruct((E, F), jnp.float32),
    )(f_ij, rcut_ij.reshape(E, 1), Wf1, bf1.reshape(1, F), Wf2,
      bf2.reshape(1, F))

    CH = E // (_NW * C)
    SUP = 25 if CH % 25 == 0 else CH
    idx_i = pairlist[0].astype(jnp.int32).reshape(_NW, CH // SUP, SUP, C)
    idx_j = pairlist[1].astype(jnp.int32).reshape(_NW, CH // SUP, SUP, C)

    NP = ((N + _NS * 128 - 1) // (_NS * 128)) * (_NS * 128)
    acc = _make_sc_scatter(NP, E, F, C)(h, wij, idx_j, idx_i)

    out = pl.pallas_call(
        _out_body,
        grid=(N // BN,),
        in_specs=[
            pl.BlockSpec((_NC, BN, F), lambda i: (0, i, 0)),
            pl.BlockSpec((F, D), lambda i: (0, 0)),
            pl.BlockSpec((1, D), lambda i: (0, 0)),
            pl.BlockSpec((D, D), lambda i: (0, 0)),
            pl.BlockSpec((1, D), lambda i: (0, 0)),
        ],
        out_specs=pl.BlockSpec((BN, D), lambda i: (i, 0)),
        out_shape=jax.ShapeDtypeStruct((N, D), jnp.float32),
    )(acc, Wo1, bo1.reshape(1, D), Wo2, bo2.reshape(1, D))

    return out
